# baseline (speedup 1.0000x reference)
import jax
import jax.numpy as jnp
from jax.experimental import pallas as pl
from jax.experimental.pallas import tpu as pltpu


def _gcn_kernel(adj_ref, x_ref, w_ref, b_ref, o_ref):
    xw = jnp.dot(
        x_ref[...].astype(jnp.bfloat16),
        w_ref[...].astype(jnp.bfloat16),
        preferred_element_type=jnp.float32,
    )
    acc = jax.lax.dot_general(
        adj_ref[...].astype(jnp.bfloat16),
        xw.astype(jnp.bfloat16),
        (((1,), (0,)), ((), ())),
        preferred_element_type=jnp.float32,
    )
    o_ref[...] = acc + b_ref[...]


def kernel(x, adj, w, b):
    n, f = x.shape
    h = w.shape[1]
    bm = 400
    b2 = b.reshape(1, h)
    out = pl.pallas_call(
        _gcn_kernel,
        grid=(pl.cdiv(n, bm),),
        in_specs=[
            pl.BlockSpec((bm, n), lambda i: (i, 0)),
            pl.BlockSpec((n, f), lambda i: (0, 0)),
            pl.BlockSpec((f, h), lambda i: (0, 0)),
            pl.BlockSpec((1, h), lambda i: (0, 0)),
        ],
        out_specs=pl.BlockSpec((bm, h), lambda i: (i, 0)),
        out_shape=jax.ShapeDtypeStruct((n, h), jnp.float32),
        compiler_params=pltpu.CompilerParams(
            dimension_semantics=("parallel",),
        ),
    )(adj, x, w, b2)
    return out


# speedup vs baseline: 1.0032x; 1.0032x over previous
import jax
import jax.numpy as jnp
from jax.experimental import pallas as pl
from jax.experimental.pallas import tpu as pltpu

_S = 5
_BM = 80


def _xw_kernel(x_ref, w_ref, o_ref):
    o_ref[...] = jnp.dot(
        x_ref[...], w_ref[...], preferred_element_type=jnp.float32
    )


def _spmm_kernel(*refs):
    adj_refs = refs[:_S]
    xw_ref, b_ref, o_ref = refs[_S], refs[_S + 1], refs[_S + 2]
    xw = xw_ref[...].astype(jnp.bfloat16)
    bias = b_ref[...]
    for c in range(_S):
        acc = jax.lax.dot_general(
            adj_refs[c][...].astype(jnp.bfloat16),
            xw,
            (((1,), (0,)), ((), ())),
            preferred_element_type=jnp.float32,
        )
        o_ref[c * _BM:(c + 1) * _BM, :] = acc + bias


def kernel(x, adj, w, b):
    n, f = x.shape
    h = w.shape[1]
    xw = pl.pallas_call(
        _xw_kernel,
        out_shape=jax.ShapeDtypeStruct((n, h), jnp.float32),
    )(x, w)

    b2 = b.reshape(1, h)
    rows_per_step = _S * _BM

    def adj_map(c):
        return lambda i: (i * _S + c, 0)

    out = pl.pallas_call(
        _spmm_kernel,
        grid=(n // rows_per_step,),
        in_specs=[pl.BlockSpec((_BM, n), adj_map(c)) for c in range(_S)]
        + [
            pl.BlockSpec((n, h), lambda i: (0, 0)),
            pl.BlockSpec((1, h), lambda i: (0, 0)),
        ],
        out_specs=pl.BlockSpec((rows_per_step, h), lambda i: (i, 0)),
        out_shape=jax.ShapeDtypeStruct((n, h), jnp.float32),
        compiler_params=pltpu.CompilerParams(
            dimension_semantics=("parallel",),
        ),
    )(adj, adj, adj, adj, adj, xw, b2)
    return out


# speedup vs baseline: 1.0070x; 1.0038x over previous
import jax
import jax.numpy as jnp
from jax.experimental import pallas as pl
from jax.experimental.pallas import tpu as pltpu

_S = 5
_BM = 80


def _xw_kernel(x_ref, w_ref, o_ref):
    o_ref[...] = jnp.dot(
        x_ref[...], w_ref[...], preferred_element_type=jnp.float32
    )


def _spmm_kernel(*refs):
    adj_refs = refs[:_S]
    xw_ref, b_ref, o_ref = refs[_S], refs[_S + 1], refs[_S + 2]
    xw = xw_ref[...]
    bias = b_ref[...]
    for c in range(_S):
        acc = jax.lax.dot_general(
            adj_refs[c][...],
            xw,
            (((1,), (0,)), ((), ())),
            preferred_element_type=jnp.float32,
        )
        o_ref[c * _BM:(c + 1) * _BM, :] = acc + bias


def kernel(x, adj, w, b):
    n, f = x.shape
    h = w.shape[1]
    xw = pl.pallas_call(
        _xw_kernel,
        out_shape=jax.ShapeDtypeStruct((n, h), jnp.float32),
    )(x, w)

    b2 = b.reshape(1, h)
    rows_per_step = _S * _BM

    def adj_map(c):
        return lambda i: (i * _S + c, 0)

    out = pl.pallas_call(
        _spmm_kernel,
        grid=(n // rows_per_step,),
        in_specs=[pl.BlockSpec((_BM, n), adj_map(c)) for c in range(_S)]
        + [
            pl.BlockSpec((n, h), lambda i: (0, 0)),
            pl.BlockSpec((1, h), lambda i: (0, 0)),
        ],
        out_specs=pl.BlockSpec((rows_per_step, h), lambda i: (i, 0)),
        out_shape=jax.ShapeDtypeStruct((n, h), jnp.float32),
        compiler_params=pltpu.CompilerParams(
            dimension_semantics=("parallel",),
        ),
    )(adj, adj, adj, adj, adj, xw, b2)
    return out
